# baseline (speedup 1.0000x reference)
import functools

import jax
import jax.numpy as jnp
from jax import lax
from jax.experimental import pallas as pl
from jax.experimental.pallas import tpu as pltpu
from jax.experimental.pallas import tpu_sc as plsc

VOCAB = 1000
B = 4096
S = 20
CB = 4

_info = plsc.get_sparse_core_info()
NC, NS = _info.num_cores, _info.num_subcores
NW = NC * NS
BPW = B // NW
NCHUNK = BPW // CB


def _sc_kernel(x_hbm, zeros_hbm, out_hbm, zbuf, sem):
    wid = lax.axis_index("s") * NC + lax.axis_index("c")
    pltpu.sync_copy(zeros_hbm, zbuf)

    copies = []
    for c in range(NCHUNK):
        b0 = wid * BPW + c * CB
        copies.append(pltpu.make_async_copy(zbuf, out_hbm.at[pl.ds(b0, CB)], sem))
    for cp in copies:
        cp.start()
    for cp in copies:
        cp.wait()


def kernel(x):
    xi = x.astype(jnp.int32).reshape(B * S)
    zeros = jnp.zeros((CB, S, VOCAB), jnp.float32)
    mesh = plsc.VectorSubcoreMesh(core_axis_name="c", subcore_axis_name="s")
    k = functools.partial(
        pl.kernel,
        out_type=jax.ShapeDtypeStruct((B, S, VOCAB), jnp.float32),
        mesh=mesh,
        scratch_types=[
            pltpu.VMEM((CB, S, VOCAB), jnp.float32),
            pltpu.SemaphoreType.DMA,
        ],
    )(_sc_kernel)
    return k(xi, zeros)


# speedup vs baseline: 1.0777x; 1.0777x over previous
import jax
import jax.numpy as jnp
from jax.experimental import pallas as pl
from jax.experimental.pallas import tpu as pltpu

VOCAB = 1000
BATCH_BLOCK = 128


def _onehot_block(x_ref, o_ref):
    bb, s = x_ref.shape
    sp, vp = o_ref.shape[1], o_ref.shape[2]
    idx = x_ref[...].reshape(bb, s, 1)
    idx = jnp.pad(idx, ((0, 0), (0, sp - s), (0, 0)), constant_values=-1)
    iota = jax.lax.broadcasted_iota(jnp.int32, (bb, sp, vp), 2)
    o_ref[...] = (idx == iota).astype(jnp.float32)


def kernel(x):
    B, S = x.shape
    xi = x.astype(jnp.int32)
    nblocks = B // BATCH_BLOCK
    out = pl.pallas_call(
        _onehot_block,
        grid=(nblocks,),
        in_specs=[pl.BlockSpec((BATCH_BLOCK, S), lambda i: (i, 0))],
        out_specs=pl.BlockSpec((BATCH_BLOCK, 24, 1024), lambda i: (i, 0, 0)),
        out_shape=jax.ShapeDtypeStruct((B, S, VOCAB), jnp.float32),
    )(xi)
    return out


# speedup vs baseline: 1.3092x; 1.2148x over previous
import jax
import jax.numpy as jnp
from jax.experimental import pallas as pl
from jax.experimental.pallas import tpu as pltpu

VOCAB = 1000
BATCH_BLOCK = 128
S_PAD = 24
V_PAD = 1024


def _onehot_block(x_ref, o_ref):
    bb, s = x_ref.shape
    idx = x_ref[...].reshape(bb, s, 1)
    idx = jnp.pad(idx, ((0, 0), (0, S_PAD - s), (0, 0)), constant_values=-1)
    iota = jax.lax.broadcasted_iota(jnp.int32, (bb, S_PAD, V_PAD), 2)
    o_ref[...] = (idx == iota).astype(jnp.float32)


def kernel(x):
    B, S = x.shape
    xi = x.astype(jnp.int32)
    nblocks = B // BATCH_BLOCK
    padded = pl.pallas_call(
        _onehot_block,
        grid=(nblocks,),
        in_specs=[pl.BlockSpec((BATCH_BLOCK, S), lambda i: (i, 0))],
        out_specs=pl.BlockSpec((BATCH_BLOCK, S_PAD, V_PAD), lambda i: (i, 0, 0)),
        out_shape=jax.ShapeDtypeStruct((B, S_PAD, V_PAD), jnp.float32),
    )(xi)
    return padded[:, :S, :VOCAB]


# speedup vs baseline: 1.3109x; 1.0013x over previous
import jax
import jax.numpy as jnp
from jax.experimental import pallas as pl
from jax.experimental.pallas import tpu as pltpu

VOCAB = 1000
BATCH_BLOCK = 256
S_PAD = 24
V_PAD = 1024


def _onehot_block(x_ref, o_ref):
    bb, s = x_ref.shape
    idx = x_ref[...].reshape(bb, s, 1)
    idx = jnp.pad(idx, ((0, 0), (0, S_PAD - s), (0, 0)), constant_values=-1)
    iota = jax.lax.broadcasted_iota(jnp.int32, (bb, S_PAD, V_PAD), 2)
    o_ref[...] = (idx == iota).astype(jnp.float32)


def kernel(x):
    B, S = x.shape
    xi = x.astype(jnp.int32)
    nblocks = B // BATCH_BLOCK
    padded = pl.pallas_call(
        _onehot_block,
        grid=(nblocks,),
        in_specs=[pl.BlockSpec((BATCH_BLOCK, S), lambda i: (i, 0))],
        out_specs=pl.BlockSpec((BATCH_BLOCK, S_PAD, V_PAD), lambda i: (i, 0, 0)),
        out_shape=jax.ShapeDtypeStruct((B, S_PAD, V_PAD), jnp.float32),
    )(xi)
    return padded[:, :S, :VOCAB]
